# baseline (speedup 1.0000x reference)
import functools

import jax
import jax.numpy as jnp
from jax import lax
from jax.experimental import pallas as pl
from jax.experimental.pallas import tpu as pltpu
from jax.experimental.pallas import tpu_sc as plsc

VOCAB = 1000000
EMB = 16
HID = 32
BATCH = 16384

NC = 2
NS = 16
NW = NC * NS
NIDX = 4 * BATCH
B_PER_W = NIDX // NW
CH = 256
NCH = B_PER_W // CH


def _make_gather():
    mesh = plsc.VectorSubcoreMesh(
        core_axis_name="c", subcore_axis_name="s", num_cores=NC, num_subcores=NS
    )

    @functools.partial(
        pl.kernel,
        mesh=mesh,
        compiler_params=pltpu.CompilerParams(use_tc_tiling_on_sc=True),
        out_type=jax.ShapeDtypeStruct((NIDX, EMB), jnp.float32),
        scratch_types=[
            pltpu.VMEM((NCH, CH), jnp.int32),
            pltpu.SemaphoreType.DMA,
            pltpu.SemaphoreType.DMA,
        ],
    )
    def gather_kernel(idx_hbm, table_hbm, out_hbm, idx_v, sem_i, sem_r):
        wid = lax.axis_index("s") * NC + lax.axis_index("c")
        base = wid * B_PER_W
        pltpu.async_copy(idx_hbm.at[wid], idx_v, sem_i).wait()

        def chunk_body(c, carry):
            obase = base + c * CH

            def group_body(g, carry2):
                vec = idx_v[c, pl.ds(g * 16, 16)]
                gbase = obase + g * 16
                for j in range(16):
                    pltpu.async_copy(
                        table_hbm.at[pl.ds(vec[j], 1)],
                        out_hbm.at[pl.ds(gbase + j, 1)],
                        sem_r,
                    )
                return carry2

            lax.fori_loop(0, CH // 16, group_body, 0)

            def drain_body(i, carry2):
                pltpu.make_async_copy(
                    table_hbm.at[pl.ds(0, 1)],
                    out_hbm.at[pl.ds(obase + i, 1)],
                    sem_r,
                ).wait()
                return carry2

            lax.fori_loop(0, CH, drain_body, 0)
            return carry

        lax.fori_loop(0, NCH, chunk_body, 0)

    return gather_kernel


_gather = _make_gather()

BB = 2048
GRID = BATCH // BB


def _mlp_body(e_ref, w_ref, b_ref, ws_ref, bs_ref, o_ref):
    acc = jnp.zeros((BB, HID), jnp.float32)
    for i in range(4):
        h = jnp.dot(e_ref[i], w_ref[i], preferred_element_type=jnp.float32) + b_ref[i]
        acc = acc + jnp.maximum(h, 0.0)
    o_ref[...] = (
        jnp.dot(acc, ws_ref[...], preferred_element_type=jnp.float32) + bs_ref[...]
    )


def kernel(x, table, w1, b1, w2, b2, w3, b3, w4, b4, ws, bs):
    idx = jnp.stack([x[:, 0], x[:, 1], x[:, 3], x[:, 4]], axis=0)
    idx = idx.reshape(NW, NCH, CH)

    rows = _gather(idx, table)
    e = rows.reshape(4, BATCH, EMB)

    w_all = jnp.stack([w1, w2, w3, w4], axis=0)
    b_all = jnp.stack([b1, b2, b3, b4], axis=0)[:, None, :]

    out = pl.pallas_call(
        _mlp_body,
        grid=(GRID,),
        in_specs=[
            pl.BlockSpec((4, BB, EMB), lambda i: (0, i, 0)),
            pl.BlockSpec((4, EMB, HID), lambda i: (0, 0, 0)),
            pl.BlockSpec((4, 1, HID), lambda i: (0, 0, 0)),
            pl.BlockSpec((HID, EMB), lambda i: (0, 0)),
            pl.BlockSpec((1, EMB), lambda i: (0, 0)),
        ],
        out_specs=pl.BlockSpec((BB, EMB), lambda i: (i, 0)),
        out_shape=jax.ShapeDtypeStruct((BATCH, EMB), jnp.float32),
    )(e, w_all, b_all, ws, bs[None, :])
    return out


# speedup vs baseline: 2.7316x; 2.7316x over previous
import functools

import jax
import jax.numpy as jnp
from jax import lax
from jax.experimental import pallas as pl
from jax.experimental.pallas import tpu as pltpu
from jax.experimental.pallas import tpu_sc as plsc

VOCAB = 1000000
EMB = 16
HID = 32
BATCH = 16384

NC = 2
NS = 16
NW = NC * NS
NIDX = 4 * BATCH
B_PER_W = NIDX // NW
CHUNK = 128
NCHUNK = B_PER_W // CHUNK

PK = 128 // EMB
QTOT = NIDX // PK
QC = BATCH // PK


def _make_gather():
    mesh = plsc.VectorSubcoreMesh(
        core_axis_name="c", subcore_axis_name="s", num_cores=NC, num_subcores=NS
    )

    @functools.partial(
        pl.kernel,
        mesh=mesh,
        compiler_params=pltpu.CompilerParams(use_tc_tiling_on_sc=False),
        out_type=jax.ShapeDtypeStruct((NIDX, EMB), jnp.float32),
        scratch_types=[
            pltpu.VMEM((NCHUNK, CHUNK), jnp.int32),
            pltpu.VMEM((B_PER_W, EMB), jnp.float32),
            pltpu.SemaphoreType.DMA,
        ],
    )
    def gather_kernel(idx_hbm, table_hbm, out_hbm, idx_v, rows_v, sem):
        wid = lax.axis_index("s") * NC + lax.axis_index("c")
        base = wid * B_PER_W
        pltpu.sync_copy(idx_hbm.at[wid], idx_v)
        copies = []
        for j in range(NCHUNK):
            copies.append(
                pltpu.async_copy(
                    table_hbm.at[idx_v.at[j]],
                    rows_v.at[pl.ds(j * CHUNK, CHUNK)],
                    sem,
                )
            )
        for c in copies:
            c.wait()
        pltpu.sync_copy(rows_v, out_hbm.at[pl.ds(base, B_PER_W)])

    return gather_kernel


_gather = _make_gather()

BQ = 512
GRID = QC // BQ


def _mlp_body(e_ref, w_ref, b_ref, ws_ref, bs_ref, o_ref):
    acc = jnp.zeros((BQ, PK * HID), jnp.float32)
    for c in range(4):
        h = jnp.dot(e_ref[c], w_ref[c], preferred_element_type=jnp.float32) + b_ref[c]
        acc = acc + jnp.maximum(h, 0.0)
    o_ref[...] = (
        jnp.dot(acc, ws_ref[...], preferred_element_type=jnp.float32) + bs_ref[...]
    )


def kernel(x, table, w1, b1, w2, b2, w3, b3, w4, b4, ws, bs):
    idx = jnp.stack([x[:, 0], x[:, 1], x[:, 3], x[:, 4]], axis=0)
    idx = idx.reshape(NW, NCHUNK, CHUNK)

    rows = _gather(idx, table)
    e_pk = rows.reshape(4, QC, PK * EMB)

    eye = jnp.eye(PK, dtype=jnp.float32)
    w_bd = jnp.stack(
        [jnp.kron(eye, w) for w in (w1, w2, w3, w4)], axis=0
    )
    b_bd = jnp.stack(
        [jnp.tile(b, PK) for b in (b1, b2, b3, b4)], axis=0
    )[:, None, :]
    ws_bd = jnp.kron(eye, ws)
    bs_bd = jnp.tile(bs, PK)[None, :]

    out = pl.pallas_call(
        _mlp_body,
        grid=(GRID,),
        in_specs=[
            pl.BlockSpec((4, BQ, PK * EMB), lambda i: (0, i, 0)),
            pl.BlockSpec((4, PK * EMB, PK * HID), lambda i: (0, 0, 0)),
            pl.BlockSpec((4, 1, PK * HID), lambda i: (0, 0, 0)),
            pl.BlockSpec((PK * HID, PK * EMB), lambda i: (0, 0)),
            pl.BlockSpec((1, PK * EMB), lambda i: (0, 0)),
        ],
        out_specs=pl.BlockSpec((BQ, PK * EMB), lambda i: (i, 0)),
        out_shape=jax.ShapeDtypeStruct((QC, PK * EMB), jnp.float32),
    )(e_pk, w_bd, b_bd, ws_bd, bs_bd)
    return out.reshape(BATCH, EMB)
